# Initial kernel scaffold; baseline (speedup 1.0000x reference)
import jax
import jax.numpy as jnp
from jax.experimental import pallas as pl


def kernel(feat0, feat1, feat2, Wb0, bb0, Wp0, bp0, Wb1, bb1, Wp1, bp1, Wb2, bb2, Wp2, bp2):
    raise NotImplementedError("write your pallas kernel here")


# speedup vs baseline: 9.5627x; 9.5627x over previous
import functools

import jax
import jax.numpy as jnp
from jax import lax
from jax.experimental import pallas as pl
from jax.experimental.pallas import tpu as pltpu
from jax.experimental.pallas import tpu_sc as plsc

_NC = 4
_NA = 3
_NATTR = 9
_FEAT_HW = [16, 32, 64]
_STRIDES = [32.0, 16.0, 8.0]
_ANCHORS = [[[116, 90], [156, 198], [373, 326]],
            [[30, 61], [62, 45], [59, 119]],
            [[10, 13], [16, 30], [33, 23]]]
_SCORE_THR = 0.05
_IOU_THR = 0.45
_NMS_PRE = 1000
_MAX_PER_IMG = 100
_SCALE_OFF = [0, 768, 3840]

_NKEYS = 64512
_NW = 32
_CHUNK = _NKEYS // _NW
_CAP = 256
_NCAND = 1024
_NSORT = 1024


def _conv_body(x_ref, w_ref, b_ref, o_ref):
    y = jnp.dot(x_ref[...], w_ref[...], preferred_element_type=jnp.float32)
    y = y + b_ref[...]
    o_ref[...] = jnp.where(y > 0, y, 0.1 * y)


def _conv_leaky(x9, w9, b, m_tile):
    hw, k = x9.shape
    oc = w9.shape[1]
    return pl.pallas_call(
        _conv_body,
        grid=(hw // m_tile,),
        in_specs=[
            pl.BlockSpec((m_tile, k), lambda i: (i, 0)),
            pl.BlockSpec((k, oc), lambda i: (0, 0)),
            pl.BlockSpec((1, oc), lambda i: (0, 0)),
        ],
        out_specs=pl.BlockSpec((m_tile, oc), lambda i: (i, 0)),
        out_shape=jax.ShapeDtypeStruct((hw, oc), jnp.float32),
    )(x9, w9, b.reshape(1, -1))


def _decode_body(y_ref, wp_ref, bp_ref, box_ref, key_ref, *, scale):
    W = _FEAT_HW[scale]
    stride = _STRIDES[scale]
    p = jnp.dot(y_ref[...], wp_ref[...],
                preferred_element_type=jnp.float32) + bp_ref[...]
    n = y_ref.shape[0]
    rows = jax.lax.broadcasted_iota(jnp.int32, (n, 1), 0)
    gy = (rows // W).astype(jnp.float32)
    gx = (rows % W).astype(jnp.float32)
    boxes = []
    keys = []
    for a in range(_NA):
        tx = p[:, a * _NATTR + 0:a * _NATTR + 1]
        ty = p[:, a * _NATTR + 1:a * _NATTR + 2]
        tw = p[:, a * _NATTR + 2:a * _NATTR + 3]
        th = p[:, a * _NATTR + 3:a * _NATTR + 4]
        obj = p[:, a * _NATTR + 4:a * _NATTR + 5]
        cls = p[:, a * _NATTR + 5:a * _NATTR + 9]
        cx = (jax.nn.sigmoid(tx) + gx) * stride
        cy = (jax.nn.sigmoid(ty) + gy) * stride
        bw = jnp.exp(jnp.clip(tw, -10.0, 10.0)) * float(_ANCHORS[scale][a][0])
        bh = jnp.exp(jnp.clip(th, -10.0, 10.0)) * float(_ANCHORS[scale][a][1])
        boxes += [cx - bw * 0.5, cy - bh * 0.5, cx + bw * 0.5, cy + bh * 0.5]
        sc = jax.nn.sigmoid(obj) * jax.nn.sigmoid(cls)
        sidx = (_SCALE_OFF[scale] + rows * _NA + a) * _NC + \
            jax.lax.broadcasted_iota(jnp.int32, (n, _NC), 1)
        inv = _SCORE_THR - sidx.astype(jnp.float32) * 1e-7
        keys.append(jnp.where(sc > _SCORE_THR, sc, inv))
    box_ref[...] = jnp.concatenate(boxes, axis=1)
    key_ref[...] = jnp.concatenate(keys, axis=1)


def _decode(y, wp, bp, scale):
    hw, oc = y.shape
    return pl.pallas_call(
        functools.partial(_decode_body, scale=scale),
        out_shape=[
            jax.ShapeDtypeStruct((hw, 12), jnp.float32),
            jax.ShapeDtypeStruct((hw, 12), jnp.float32),
        ],
    )(y, wp, bp.reshape(1, -1))


def _im2col(x, H, W):
    c = x.shape[1]
    xp = jnp.pad(x[0], ((0, 0), (1, 1), (1, 1)))
    cols = []
    for ky in range(3):
        for kx in range(3):
            cols.append(xp[:, ky:ky + H, kx:kx + W].reshape(c, H * W).T)
    return jnp.concatenate(cols, axis=1)


def _thresh_body(k_ref, t_ref):
    keys = k_ref[...]

    def count_ge(bits):
        t = jax.lax.bitcast_convert_type(bits, jnp.float32)
        return jnp.sum((keys >= t).astype(jnp.int32))

    lo0 = jax.lax.bitcast_convert_type(jnp.float32(0.04), jnp.int32)
    hi0 = jax.lax.bitcast_convert_type(jnp.float32(1.0), jnp.int32)

    def body(_, carry):
        lo, hi = carry
        mid = (lo + hi) // 2
        ge = count_ge(mid) >= _NMS_PRE
        return jnp.where(ge, mid, lo), jnp.where(ge, hi, mid)

    lo, hi = jax.lax.fori_loop(0, 26, body, (lo0, hi0))
    t_ref[...] = jnp.full((1, 128),
                          jax.lax.bitcast_convert_type(lo, jnp.float32))


def _threshold(keys2d):
    return pl.pallas_call(
        _thresh_body,
        out_shape=jax.ShapeDtypeStruct((1, 128), jnp.float32),
    )(keys2d)


def _sc_compact(keys_hbm, thr_hbm):
    mesh = plsc.VectorSubcoreMesh(core_axis_name="c", subcore_axis_name="s")

    @functools.partial(
        pl.kernel, mesh=mesh,
        out_type=[
            jax.ShapeDtypeStruct((_NCAND,), jnp.float32),
            jax.ShapeDtypeStruct((_NCAND,), jnp.int32),
        ],
        scratch_types=[
            pltpu.VMEM((_CHUNK // 16, 16), jnp.float32),
            pltpu.VMEM((16,), jnp.float32),
            pltpu.VMEM((_CAP,), jnp.float32),
            pltpu.VMEM((_CAP,), jnp.int32),
        ],
    )
    def k(keys_h, thr_h, okey_h, oidx_h, kv, tv, ck, civ):
        wid = lax.axis_index("s") * 2 + lax.axis_index("c")
        pltpu.sync_copy(keys_h.at[wid], kv)
        pltpu.sync_copy(thr_h, tv)
        t = tv[...]
        lane = lax.iota(jnp.int32, 16)
        neg1 = jnp.full((16,), -1.0, jnp.float32)
        zero = jnp.zeros((16,), jnp.int32)
        for i in range(_CAP // 16):
            ck[pl.ds(i * 16, 16)] = neg1
            civ[pl.ds(i * 16, 16)] = zero

        cnt_vec = jnp.zeros((16,), jnp.int32)
        for i in range(_CHUNK // 16):
            v = kv[i]
            m = v >= t
            mi = m.astype(jnp.int32)
            q = cnt_vec * 16 + lane
            ok = m & (cnt_vec < _CAP // 16)
            plsc.store_scatter(ck, [q], v, mask=ok)
            gidx = (i * 16 + lane) * _NW + wid
            plsc.store_scatter(civ, [q], gidx, mask=ok)
            cnt_vec = cnt_vec + mi
        pltpu.sync_copy(ck, okey_h.at[pl.ds(wid * _CAP, _CAP)])
        pltpu.sync_copy(civ, oidx_h.at[pl.ds(wid * _CAP, _CAP)])

    return k(keys_hbm, thr_hbm)


def _sort_body(kc_ref, kr_ref, ic_ref, ir_ref, ok_ref, oi_ref):
    kr = kr_ref[...]
    ir = ir_ref[...]
    nch = 8
    rows = _NCAND // nch
    tgt = jax.lax.broadcasted_iota(jnp.int32, (1, _NSORT), 1).astype(jnp.float32)
    ok = jnp.zeros((1, _NSORT), jnp.float32)
    oi = jnp.zeros((1, _NSORT), jnp.float32)
    for c in range(nch):
        kc = kc_ref[pl.ds(c * rows, rows), :]
        ic = ic_ref[pl.ds(c * rows, rows), :]
        gt = (kr > kc).astype(jnp.float32)
        tie = ((kr == kc) & (ir < ic)).astype(jnp.float32)
        rank = jnp.sum(gt + tie, axis=1, keepdims=True)
        onehot = (rank == tgt).astype(jnp.float32)
        ok = ok + jnp.sum(onehot * kc, axis=0, keepdims=True)
        oi = oi + jnp.sum(onehot * ic, axis=0, keepdims=True)
    ok_ref[...] = ok
    oi_ref[...] = oi


def _sort(ckeys, cidx):
    kc = ckeys.reshape(_NCAND, 1)
    kr = ckeys.reshape(1, _NCAND)
    ic = cidx.astype(jnp.float32).reshape(_NCAND, 1)
    ir = cidx.astype(jnp.float32).reshape(1, _NCAND)
    return pl.pallas_call(
        _sort_body,
        out_shape=[
            jax.ShapeDtypeStruct((1, _NSORT), jnp.float32),
            jax.ShapeDtypeStruct((1, _NSORT), jnp.float32),
        ],
    )(kc, kr, ic, ir)


def _sc_gather(boxes_hbm, bidx_hbm):
    mesh = plsc.VectorSubcoreMesh(core_axis_name="c", subcore_axis_name="s")
    rows = _NSORT // _NW

    @functools.partial(
        pl.kernel, mesh=mesh,
        out_type=jax.ShapeDtypeStruct((_NSORT, 128), jnp.float32),
        scratch_types=[
            pltpu.VMEM((rows,), jnp.int32),
            pltpu.VMEM((rows, 128), jnp.float32),
            pltpu.SemaphoreType.DMA,
        ],
    )
    def k(boxes_h, bidx_h, out_h, idx_v, rows_v, sem):
        wid = lax.axis_index("s") * 2 + lax.axis_index("c")
        base = wid * rows
        pltpu.sync_copy(bidx_h.at[pl.ds(base, rows)], idx_v)
        pltpu.async_copy(boxes_h.at[idx_v], rows_v, sem).wait()
        pltpu.sync_copy(rows_v, out_h.at[pl.ds(base, rows)])

    return k(boxes_hbm, bidx_hbm)


def _row_to_col(row, c, rows):
    n = row.shape[1]
    ecol = jax.lax.broadcasted_iota(jnp.int32, (rows, n), 0) + c * rows
    erow = jax.lax.broadcasted_iota(jnp.int32, (rows, n), 1)
    onehot = (ecol == erow).astype(jnp.float32)
    return jnp.sum(onehot * row, axis=1, keepdims=True)


def _nms_body(bc_ref, br_ref, kc_ref, kr_ref, cc_ref, cr_ref,
              det_ref, lab_ref, s_ref):
    n = _NSORT
    kr = kr_ref[...]
    cr = cr_ref[...]
    posr = jax.lax.broadcasted_iota(jnp.int32, (1, n), 1).astype(jnp.float32)
    validr = (kr > _SCORE_THR) & (posr < float(_NMS_PRE))

    offr = cr * 4096.0
    bx1r = br_ref[0:1, :] + offr
    by1r = br_ref[1:2, :] + offr
    bx2r = br_ref[2:3, :] + offr
    by2r = br_ref[3:4, :] + offr
    arear = (bx2r - bx1r) * (by2r - by1r)

    nch = 8
    rows = n // nch
    for c in range(nch):
        offc = cc_ref[pl.ds(c * rows, rows), :] * 4096.0
        bx1c = bc_ref[pl.ds(c * rows, rows), 0:1] + offc
        by1c = bc_ref[pl.ds(c * rows, rows), 1:2] + offc
        bx2c = bc_ref[pl.ds(c * rows, rows), 2:3] + offc
        by2c = bc_ref[pl.ds(c * rows, rows), 3:4] + offc
        x1 = jnp.maximum(bx1c, bx1r)
        y1 = jnp.maximum(by1c, by1r)
        x2 = jnp.minimum(bx2c, bx2r)
        y2 = jnp.minimum(by2c, by2r)
        inter = jnp.maximum(x2 - x1, 0.0) * jnp.maximum(y2 - y1, 0.0)
        areac = (bx2c - bx1c) * (by2c - by1c)
        iou = inter / (areac + arear - inter + 1e-6)
        posc = c * rows + jax.lax.broadcasted_iota(jnp.int32, (rows, 1), 0).astype(jnp.float32)
        kcc = kc_ref[pl.ds(c * rows, rows), :]
        validc = (kcc > _SCORE_THR) & (posc < float(_NMS_PRE))
        sup = (iou > _IOU_THR) & (posr > posc) & validc & validr
        s_ref[pl.ds(c * rows, rows), :] = sup.astype(jnp.float32)

    valid_f = validr.astype(jnp.float32)

    def cond(carry):
        _, changed = carry
        return changed

    def body(carry):
        keep, _ = carry
        supcnt = jnp.dot(keep, s_ref[...],
                         preferred_element_type=jnp.float32)
        new = valid_f * jnp.where(supcnt > 0, 0.0, 1.0)
        changed = jnp.sum(jnp.abs(new - keep)) > 0
        return new, changed

    keep, _ = jax.lax.while_loop(cond, body, (valid_f, True))

    fin = jnp.where(keep > 0, kr, -(posr + 1.0))

    tgt = jax.lax.broadcasted_iota(jnp.int32, (1, 128), 1).astype(jnp.float32)
    acc = [jnp.zeros((1, 128), jnp.float32) for _ in range(7)]
    for c in range(nch):
        posc = c * rows + jax.lax.broadcasted_iota(jnp.int32, (rows, 1), 0).astype(jnp.float32)
        kcc = kc_ref[pl.ds(c * rows, rows), :]
        keep_c = _row_to_col(keep, c, rows)
        fin_c = jnp.where(keep_c > 0, kcc, -(posc + 1.0))
        rank = jnp.sum((fin > fin_c).astype(jnp.float32), axis=1,
                       keepdims=True)
        onehot = (rank == tgt).astype(jnp.float32)
        vals = [bc_ref[pl.ds(c * rows, rows), 0:1],
                bc_ref[pl.ds(c * rows, rows), 1:2],
                bc_ref[pl.ds(c * rows, rows), 2:3],
                bc_ref[pl.ds(c * rows, rows), 3:4],
                kcc,
                cc_ref[pl.ds(c * rows, rows), :],
                keep_c]
        for t in range(7):
            acc[t] = acc[t] + jnp.sum(onehot * vals[t], axis=0,
                                      keepdims=True)
    x1o, y1o, x2o, y2o, keyo, clso, keepo = acc
    kept = keepo > 0
    score = jnp.where(kept, keyo, 0.0)
    lab = jnp.where(kept, clso, -1.0)
    det_ref[0:1, :] = x1o
    det_ref[1:2, :] = y1o
    det_ref[2:3, :] = x2o
    det_ref[3:4, :] = y2o
    det_ref[4:5, :] = score
    lab_ref[...] = lab.astype(jnp.int32)


def _nms_final(sel, skey, scls):
    bc = sel
    br = sel.T
    kc = skey.reshape(_NSORT, 1)
    kr = skey.reshape(1, _NSORT)
    cc = scls.reshape(_NSORT, 1)
    cr = scls.reshape(1, _NSORT)
    det5, lab = pl.pallas_call(
        _nms_body,
        out_shape=[
            jax.ShapeDtypeStruct((5, 128), jnp.float32),
            jax.ShapeDtypeStruct((1, 128), jnp.int32),
        ],
        scratch_shapes=[pltpu.VMEM((_NSORT, _NSORT), jnp.float32)],
    )(bc, br, kc, kr, cc, cr)
    det = det5.T[:_MAX_PER_IMG]
    labels = lab[0, :_MAX_PER_IMG]
    return det, labels


def kernel(feat0, feat1, feat2, Wb0, bb0, Wp0, bp0, Wb1, bb1, Wp1, bp1,
           Wb2, bb2, Wp2, bp2):
    feats = [feat0, feat1, feat2]
    params = [(Wb0, bb0, Wp0, bp0), (Wb1, bb1, Wp1, bp1), (Wb2, bb2, Wp2, bp2)]
    m_tiles = [256, 1024, 1024]
    all_boxes, all_keys = [], []
    for i in range(3):
        H = W = _FEAT_HW[i]
        Wb, bb, Wp, bp = params[i]
        c = feats[i].shape[1]
        oc = Wb.shape[0]
        x9 = _im2col(feats[i], H, W)
        w9 = Wb.transpose(2, 3, 1, 0).reshape(9 * c, oc)
        y = _conv_leaky(x9, w9, bb, m_tiles[i])
        wp = Wp[:, :, 0, 0].T
        b12, k12 = _decode(y, wp, bp, i)
        all_boxes.append(b12.reshape(H * W * _NA, 4))
        all_keys.append(k12.reshape(H * W * _NA * _NC))
    bboxes = jnp.concatenate(all_boxes, 0)
    keys = jnp.concatenate(all_keys, 0)

    ckeys, cidx = jax.lax.top_k(keys, _NCAND)
    skey, sidxf = _sort(ckeys, cidx.astype(jnp.int32))
    sidx = sidxf[0].astype(jnp.int32)
    box_i = sidx // _NC
    scls = (sidx % _NC).astype(jnp.float32)
    boxes_pad = jnp.pad(bboxes, ((0, 0), (0, 124)))
    sel = _sc_gather(boxes_pad, box_i)[:, :4]
    det, labels = _nms_final(sel, skey[0], scls)
    return det, labels
